# Initial kernel scaffold; baseline (speedup 1.0000x reference)
import jax
import jax.numpy as jnp
from jax.experimental import pallas as pl


def kernel(out_labels, out_bboxes, tgt_labels, tgt_bboxes):
    raise NotImplementedError("write your pallas kernel here")


# speedup vs baseline: 2.7805x; 2.7805x over previous
import jax
import jax.numpy as jnp
from jax.experimental import pallas as pl
from jax.experimental.pallas import tpu as pltpu


_ROW_BLOCK = 480


def _cost_kernel(logits_ref, pb_ref, lab_ref, tb_ref, out_ref):
    logits = logits_ref[...]
    m = jnp.max(logits, axis=-1, keepdims=True)
    e = jnp.exp(logits - m)
    denom = jnp.sum(e, axis=-1, keepdims=True)

    C = logits.shape[-1]
    T = lab_ref.shape[-1]
    class_iota = jax.lax.broadcasted_iota(jnp.int32, (C, T), 0)
    onehot = (class_iota == lab_ref[...]).astype(jnp.float32)
    gathered = jnp.dot(e, onehot, preferred_element_type=jnp.float32)
    cost_labels = 1.0 - gathered / denom

    cx = pb_ref[:, 0:1]
    cy = pb_ref[:, 1:2]
    w = pb_ref[:, 2:3]
    h = pb_ref[:, 3:4]
    tx0 = tb_ref[0:1, :]
    ty0 = tb_ref[1:2, :]
    tx1 = tb_ref[2:3, :]
    ty1 = tb_ref[3:4, :]

    cost_bboxes = (jnp.abs(cx - tx0) + jnp.abs(cy - ty0)
                   + jnp.abs(w - tx1) + jnp.abs(h - ty1))

    px0 = cx - 0.5 * w
    py0 = cy - 0.5 * h
    px1 = cx + 0.5 * w
    py1 = cy + 0.5 * h
    area1 = (px1 - px0) * (py1 - py0)
    area2 = (tx1 - tx0) * (ty1 - ty0)

    wi = jnp.maximum(jnp.minimum(px1, tx1) - jnp.maximum(px0, tx0), 0.0)
    hi = jnp.maximum(jnp.minimum(py1, ty1) - jnp.maximum(py0, ty0), 0.0)
    inter = wi * hi
    union = area1 + area2 - inter
    iou = inter / union

    we = jnp.maximum(jnp.maximum(px1, tx1) - jnp.minimum(px0, tx0), 0.0)
    he = jnp.maximum(jnp.maximum(py1, ty1) - jnp.minimum(py0, ty0), 0.0)
    area_e = we * he
    giou = iou - (area_e - union) / area_e

    out_ref[...] = (cost_labels + 5.0 * cost_bboxes + 2.0 * (1.0 - giou))


def kernel(out_labels, out_bboxes, tgt_labels, tgt_bboxes):
    B, N, C = out_labels.shape
    T = tgt_labels.shape[0]
    BN = B * N
    RB = _ROW_BLOCK

    logits = out_labels.reshape(BN, C)
    pb = out_bboxes.reshape(BN, 4)
    lab = tgt_labels.astype(jnp.int32).reshape(1, T)
    tbT = tgt_bboxes.T

    grid = (BN // RB,)
    out = pl.pallas_call(
        _cost_kernel,
        grid=grid,
        in_specs=[
            pl.BlockSpec((RB, C), lambda i: (i, 0)),
            pl.BlockSpec((RB, 4), lambda i: (i, 0)),
            pl.BlockSpec((1, T), lambda i: (0, 0)),
            pl.BlockSpec((4, T), lambda i: (0, 0)),
        ],
        out_specs=pl.BlockSpec((RB, T), lambda i: (i, 0)),
        out_shape=jax.ShapeDtypeStruct((BN, T), jnp.float32),
        compiler_params=pltpu.CompilerParams(dimension_semantics=("parallel",)),
    )(logits, pb, lab, tbT)
    return out.reshape(B, N, T)


# speedup vs baseline: 3.7432x; 1.3462x over previous
import jax
import jax.numpy as jnp
from jax.experimental import pallas as pl
from jax.experimental.pallas import tpu as pltpu


def _cost_kernel(logits_ref, pb_ref, lab_ref, tb_ref, out_ref):
    logits = logits_ref[0]
    m = jnp.max(logits, axis=-1, keepdims=True)
    e = jnp.exp(logits - m)
    en = e / jnp.sum(e, axis=-1, keepdims=True)

    C = logits.shape[-1]
    T = lab_ref.shape[-1]
    class_iota = jax.lax.broadcasted_iota(jnp.int32, (C, T), 0)
    onehot = (class_iota == lab_ref[...]).astype(jnp.float32)
    p = jnp.dot(en, onehot, preferred_element_type=jnp.float32)

    pb = pb_ref[0]
    cx = pb[:, 0:1]
    cy = pb[:, 1:2]
    w = pb[:, 2:3]
    h = pb[:, 3:4]
    tx0 = tb_ref[0:1, :]
    ty0 = tb_ref[1:2, :]
    tx1 = tb_ref[2:3, :]
    ty1 = tb_ref[3:4, :]

    cb5 = (jnp.abs(5.0 * cx - 5.0 * tx0) + jnp.abs(5.0 * cy - 5.0 * ty0)
           + jnp.abs(5.0 * w - 5.0 * tx1) + jnp.abs(5.0 * h - 5.0 * ty1))

    px0 = cx - 0.5 * w
    py0 = cy - 0.5 * h
    px1 = cx + 0.5 * w
    py1 = cy + 0.5 * h
    area1 = (px1 - px0) * (py1 - py0)
    area2 = (tx1 - tx0) * (ty1 - ty0)

    wi = jnp.maximum(jnp.minimum(px1, tx1) - jnp.maximum(px0, tx0), 0.0)
    hi = jnp.maximum(jnp.minimum(py1, ty1) - jnp.maximum(py0, ty0), 0.0)
    inter = wi * hi
    union = (area1 + area2) - inter
    t1 = inter / union

    we = jnp.maximum(px1, tx1) - jnp.minimum(px0, tx0)
    he = jnp.maximum(py1, ty1) - jnp.minimum(py0, ty0)
    area_e = we * he
    t2 = (area_e - union) / area_e

    out_ref[0] = (3.0 - p) + cb5 + 2.0 * (t2 - t1)


def kernel(out_labels, out_bboxes, tgt_labels, tgt_bboxes):
    B, N, C = out_labels.shape
    T = tgt_labels.shape[0]

    lab = tgt_labels.astype(jnp.int32).reshape(1, T)
    tbT = tgt_bboxes.T

    return pl.pallas_call(
        _cost_kernel,
        grid=(B,),
        in_specs=[
            pl.BlockSpec((1, N, C), lambda b: (b, 0, 0)),
            pl.BlockSpec((1, N, 4), lambda b: (b, 0, 0)),
            pl.BlockSpec((1, T), lambda b: (0, 0)),
            pl.BlockSpec((4, T), lambda b: (0, 0)),
        ],
        out_specs=pl.BlockSpec((1, N, T), lambda b: (b, 0, 0)),
        out_shape=jax.ShapeDtypeStruct((B, N, T), jnp.float32),
        compiler_params=pltpu.CompilerParams(dimension_semantics=("parallel",)),
    )(out_labels, out_bboxes, lab, tbT)


# speedup vs baseline: 5.9868x; 1.5994x over previous
import jax
import jax.numpy as jnp
from jax.experimental import pallas as pl
from jax.experimental.pallas import tpu as pltpu


_NB = 60


def _cost_kernel(logits_ref, pb_ref, lab_ref, tb_ref, out_ref):
    nb, b, C = logits_ref.shape
    T = lab_ref.shape[-1]
    rows = nb * b

    logits = logits_ref[...].reshape(rows, C)
    m = jnp.max(logits, axis=-1, keepdims=True)
    e = jnp.exp(logits - m)
    en = e / jnp.sum(e, axis=-1, keepdims=True)

    class_iota = jax.lax.broadcasted_iota(jnp.int32, (C, T), 0)
    onehot = (class_iota == lab_ref[...]).astype(jnp.float32)
    p = jnp.dot(en, onehot, preferred_element_type=jnp.float32)

    pb = pb_ref[...].reshape(rows, 4)
    cx = pb[:, 0:1]
    cy = pb[:, 1:2]
    w = pb[:, 2:3]
    h = pb[:, 3:4]
    tx0 = tb_ref[0:1, :]
    ty0 = tb_ref[1:2, :]
    tx1 = tb_ref[2:3, :]
    ty1 = tb_ref[3:4, :]

    cb5 = (jnp.abs(5.0 * cx - 5.0 * tx0) + jnp.abs(5.0 * cy - 5.0 * ty0)
           + jnp.abs(5.0 * w - 5.0 * tx1) + jnp.abs(5.0 * h - 5.0 * ty1))

    px0 = cx - 0.5 * w
    py0 = cy - 0.5 * h
    px1 = cx + 0.5 * w
    py1 = cy + 0.5 * h
    area1 = (px1 - px0) * (py1 - py0)
    area2 = (tx1 - tx0) * (ty1 - ty0)

    wi = jnp.maximum(jnp.minimum(px1, tx1) - jnp.maximum(px0, tx0), 0.0)
    hi = jnp.maximum(jnp.minimum(py1, ty1) - jnp.maximum(py0, ty0), 0.0)
    inter = wi * hi
    union = (area1 + area2) - inter
    t1 = inter / union

    we = jnp.maximum(px1, tx1) - jnp.minimum(px0, tx0)
    he = jnp.maximum(py1, ty1) - jnp.minimum(py0, ty0)
    area_e = we * he
    t2 = (area_e - union) / area_e

    out = (3.0 - p) + cb5 + 2.0 * (t2 - t1)
    out_ref[...] = out.reshape(nb, b, T)


def kernel(out_labels, out_bboxes, tgt_labels, tgt_bboxes):
    B, N, C = out_labels.shape
    T = tgt_labels.shape[0]

    lt = jnp.transpose(out_labels, (1, 0, 2))
    pt = jnp.transpose(out_bboxes, (1, 0, 2))
    lab = tgt_labels.astype(jnp.int32).reshape(1, T)
    tbT = tgt_bboxes.T

    out = pl.pallas_call(
        _cost_kernel,
        grid=(N // _NB,),
        in_specs=[
            pl.BlockSpec((_NB, B, C), lambda i: (i, 0, 0)),
            pl.BlockSpec((_NB, B, 4), lambda i: (i, 0, 0)),
            pl.BlockSpec((1, T), lambda i: (0, 0)),
            pl.BlockSpec((4, T), lambda i: (0, 0)),
        ],
        out_specs=pl.BlockSpec((_NB, B, T), lambda i: (i, 0, 0)),
        out_shape=jax.ShapeDtypeStruct((N, B, T), jnp.float32),
        compiler_params=pltpu.CompilerParams(dimension_semantics=("parallel",)),
    )(lt, pt, lab, tbT)
    return jnp.transpose(out, (1, 0, 2))


# speedup vs baseline: 5.9985x; 1.0019x over previous
import jax
import jax.numpy as jnp
from jax.experimental import pallas as pl
from jax.experimental.pallas import tpu as pltpu


_NB = 60


def _cost_kernel(logits_ref, pb_ref, lab_ref, tb_ref, out_ref):
    nb, b, C = logits_ref.shape
    T = lab_ref.shape[-1]
    rows = nb * b

    logits = logits_ref[...].reshape(rows, C)
    m = jnp.max(logits, axis=-1, keepdims=True)
    e = jnp.exp(logits - m)
    en = e / (-jnp.sum(e, axis=-1, keepdims=True))

    class_iota = jax.lax.broadcasted_iota(jnp.int32, (C, T), 0)
    onehot = (class_iota == lab_ref[...]).astype(jnp.float32)
    np_ = jnp.dot(en, onehot, preferred_element_type=jnp.float32)

    pb = pb_ref[...].reshape(rows, 4)
    cx = pb[:, 0:1]
    cy = pb[:, 1:2]
    w = pb[:, 2:3]
    h = pb[:, 3:4]
    tx0 = tb_ref[0:1, :]
    ty0 = tb_ref[1:2, :]
    tx1 = tb_ref[2:3, :]
    ty1 = tb_ref[3:4, :]

    cb5 = (jnp.abs(5.0 * cx - 5.0 * tx0) + jnp.abs(5.0 * cy - 5.0 * ty0)
           + jnp.abs(5.0 * w - 5.0 * tx1) + jnp.abs(5.0 * h - 5.0 * ty1))

    px0 = cx - 0.5 * w
    py0 = cy - 0.5 * h
    px1 = cx + 0.5 * w
    py1 = cy + 0.5 * h
    pw = px1 - px0
    ph = py1 - py0
    tw = tx1 - tx0
    th = ty1 - ty0
    area1 = pw * ph
    area2 = tw * th

    dw = jnp.minimum(px1, tx1) - jnp.maximum(px0, tx0)
    dh = jnp.minimum(py1, ty1) - jnp.maximum(py0, ty0)
    inter = jnp.maximum(dw, 0.0) * jnp.maximum(dh, 0.0)
    union = (area1 + area2) - inter
    t1 = inter / union

    we = (pw + tw) - dw
    he = (ph + th) - dh
    area_e = we * he
    t2 = (area_e - union) / area_e

    d = t2 - t1
    out = (cb5 + np_) + ((d + d) + 3.0)
    out_ref[...] = out.reshape(nb, b, T)


def kernel(out_labels, out_bboxes, tgt_labels, tgt_bboxes):
    B, N, C = out_labels.shape
    T = tgt_labels.shape[0]

    lt = jnp.transpose(out_labels, (1, 0, 2))
    pt = jnp.transpose(out_bboxes, (1, 0, 2))
    lab = tgt_labels.astype(jnp.int32).reshape(1, T)
    tbT = tgt_bboxes.T

    out = pl.pallas_call(
        _cost_kernel,
        grid=(N // _NB,),
        in_specs=[
            pl.BlockSpec((_NB, B, C), lambda i: (i, 0, 0)),
            pl.BlockSpec((_NB, B, 4), lambda i: (i, 0, 0)),
            pl.BlockSpec((1, T), lambda i: (0, 0)),
            pl.BlockSpec((4, T), lambda i: (0, 0)),
        ],
        out_specs=pl.BlockSpec((_NB, B, T), lambda i: (i, 0, 0)),
        out_shape=jax.ShapeDtypeStruct((N, B, T), jnp.float32),
        compiler_params=pltpu.CompilerParams(dimension_semantics=("parallel",)),
    )(lt, pt, lab, tbT)
    return jnp.transpose(out, (1, 0, 2))
